# baseline (speedup 1.0000x reference)
import functools

import jax
import jax.numpy as jnp
from jax import lax
from jax.experimental import pallas as pl
from jax.experimental.pallas import tpu as pltpu
from jax.experimental.pallas import tpu_sc as plsc

_SC_PARAMS = pltpu.CompilerParams(needs_layout_passes=False)

E = 64
K = 2
D = 768
H = 768
T = 2048
TILE = 128
BUF = T * K + E * TILE
NT = BUF // TILE
CH = 512
BALANCE_W = 0.01

NC = 2
NS = 16
NW = NC * NS
ROWS_W = BUF // NW
GCHUNK = 64
TOK_W = T // NW


def _shift_lanes(v, sh):
    return jnp.concatenate([jnp.zeros((1, sh), v.dtype), v[:, : E - sh]], axis=1)


def _cv_sq(v):
    n = v.shape[1]
    mean = jnp.sum(v, axis=1, keepdims=True) / n
    var = jnp.sum((v - mean) ** 2, axis=1, keepdims=True) / (n - 1)
    return var / (mean * mean + 1e-10)


def _gate_body(x_ref, wg1_ref, wg2_ref, tril_ref,
               d0_ref, d1_ref, s0_ref, s1_ref,
               load_ref, imp_ref, loss_ref, te_ref, na_ref):
    xv = x_ref[...]
    a = jnp.tanh(jnp.dot(xv, wg1_ref[...],
                         preferred_element_type=jnp.float32))
    logits = jnp.dot(a, wg2_ref[...], preferred_element_type=jnp.float32)

    iota_f = lax.broadcasted_iota(jnp.int32, (T, E), 1).astype(jnp.float32)
    m1 = jnp.max(logits, axis=1, keepdims=True)
    i1 = jnp.min(jnp.where(logits == m1, iota_f, float(E)), axis=1,
                 keepdims=True)
    sel1 = iota_f == i1
    masked = jnp.where(sel1, -jnp.inf, logits)
    m2 = jnp.max(masked, axis=1, keepdims=True)
    i2 = jnp.min(jnp.where(masked == m2, iota_f, float(E)), axis=1,
                 keepdims=True)
    sel2 = iota_f == i2

    z = jnp.exp(m2 - m1)
    s0 = 1.0 / (1.0 + z)
    s1 = z / (1.0 + z)

    sel1f = sel1.astype(jnp.float32)
    sel2f = sel2.astype(jnp.float32)
    pair_cnt = sel1f + sel2f
    counts = jnp.sum(pair_cnt, axis=0, keepdims=True)
    imp = jnp.sum(sel1f * s0 + sel2f * s1, axis=0, keepdims=True)

    trilv = tril_ref[...]
    carry = jnp.zeros((1, E), jnp.float32)
    chunks = []
    for k in range(T // CH):
        ak = pair_cnt[k * CH:(k + 1) * CH, :]
        ck = jax.lax.dot_general(trilv, ak.astype(jnp.bfloat16),
                                 (((1,), (0,)), ((), ())),
                                 preferred_element_type=jnp.float32) + carry
        chunks.append(ck)
        carry = carry + jnp.sum(ak, axis=0, keepdims=True)
    csum = jnp.concatenate(chunks, axis=0)

    rank0 = jnp.sum(jnp.where(sel1, csum, 0.0), axis=1, keepdims=True)
    rank1 = jnp.sum(jnp.where(sel2, csum, 0.0), axis=1, keepdims=True)

    padded = jnp.ceil(counts / TILE) * TILE
    inc = padded
    for sh in (1, 2, 4, 8, 16, 32):
        inc = inc + _shift_lanes(inc, sh)
    pstart = inc - padded

    pstart_b = jnp.broadcast_to(pstart, (T, E))
    p0 = jnp.sum(jnp.where(sel1, pstart_b, 0.0), axis=1, keepdims=True)
    p1 = jnp.sum(jnp.where(sel2, pstart_b, 0.0), axis=1, keepdims=True)
    d0_ref[...] = (p0 + rank0).astype(jnp.int32)
    d1_ref[...] = (p1 + rank1).astype(jnp.int32)
    s0_ref[...] = s0
    s1_ref[...] = s1
    load_ref[...] = counts
    imp_ref[...] = imp

    total = jnp.sum(padded, axis=1, keepdims=True)
    na_ref[...] = (total / TILE).astype(jnp.int32)

    starts = (lax.broadcasted_iota(jnp.int32, (NT, E), 0)
              .astype(jnp.float32) * TILE)
    ps_nt = jnp.broadcast_to(pstart, (NT, E))
    te = jnp.sum((ps_nt <= starts).astype(jnp.float32), axis=1,
                 keepdims=True) - 1.0
    te_ref[...] = te.astype(jnp.int32)

    loss_ref[...] = BALANCE_W * (_cv_sq(imp) + _cv_sq(counts))


def _run_gate(xt, Wg1, Wg2, tril):
    f32 = jnp.float32
    outs = pl.pallas_call(
        _gate_body,
        out_shape=[
            jax.ShapeDtypeStruct((T, 1), jnp.int32),
            jax.ShapeDtypeStruct((T, 1), jnp.int32),
            jax.ShapeDtypeStruct((T, 1), f32),
            jax.ShapeDtypeStruct((T, 1), f32),
            jax.ShapeDtypeStruct((1, E), f32),
            jax.ShapeDtypeStruct((1, E), f32),
            jax.ShapeDtypeStruct((1, 1), f32),
            jax.ShapeDtypeStruct((NT, 1), jnp.int32),
            jax.ShapeDtypeStruct((1, 1), jnp.int32),
        ],
    )(xt, Wg1, Wg2, tril)
    return outs


def _dispatch_body(d0_hbm, d1_hbm, s0_hbm, s1_hbm, xt_hbm, na_hbm,
                   sco_hbm, xs_hbm,
                   d_v, s_v, src_v, sco_v, rows0_v, rows1_v,
                   sg0, sg1, sw0, sw1, ssco, na_v):
    wid = lax.axis_index("s") * NC + lax.axis_index("c")
    base = wid * ROWS_W
    iota16 = lax.broadcasted_iota(jnp.int32, (16,), 0)
    z16f = jnp.zeros((16,), jnp.float32)

    def init_body(j, _):
        src_v[pl.ds(j * 16, 16)] = (base + iota16 + j * 16) & (T - 1)
        sco_v[pl.ds(j * 16, 16)] = z16f
        return 0

    lax.fori_loop(0, ROWS_W // 16, init_body, 0)

    def scatter_pass(dref, sref):
        pltpu.sync_copy(dref, d_v)
        pltpu.sync_copy(sref, s_v)

        def body(i, _):
            idx = d_v[pl.ds(i * 16, 16)]
            loc = idx - base
            mask = (idx >= base) & (idx < base + ROWS_W)
            plsc.store_scatter(src_v, [loc], iota16 + i * 16, mask=mask)
            plsc.store_scatter(sco_v, [loc], s_v[pl.ds(i * 16, 16)],
                               mask=mask)
            return 0

        lax.fori_loop(0, T // 16, body, 0)

    scatter_pass(d0_hbm, s0_hbm)
    scatter_pass(d1_hbm, s1_hbm)
    wsco = pltpu.async_copy(sco_v, sco_hbm.at[pl.ds(base, ROWS_W)], ssco)

    nch = ROWS_W // GCHUNK
    bufs = (rows0_v, rows1_v)
    gsems = (sg0, sg1)
    wsems = (sw0, sw1)
    gathers = [None] * nch
    writes = [None] * nch
    for c in range(nch):
        b = c % 2
        if c >= 2:
            writes[c - 2].wait()
        gathers[c] = pltpu.async_copy(
            xt_hbm.at[src_v.at[pl.ds(c * GCHUNK, GCHUNK)]], bufs[b], gsems[b])
        if c >= 1:
            gathers[c - 1].wait()
            writes[c - 1] = pltpu.async_copy(
                bufs[(c - 1) % 2],
                xs_hbm.at[pl.ds(base + (c - 1) * GCHUNK, GCHUNK)],
                wsems[(c - 1) % 2])
    gathers[nch - 1].wait()
    writes[nch - 1] = pltpu.async_copy(
        bufs[(nch - 1) % 2],
        xs_hbm.at[pl.ds(base + (nch - 1) * GCHUNK, GCHUNK)],
        wsems[(nch - 1) % 2])
    for c in range(max(0, nch - 2), nch):
        writes[c].wait()
    wsco.wait()


def _run_dispatch(xt, d0, d1, s0, s1, nact):
    mesh = plsc.VectorSubcoreMesh(core_axis_name="c", subcore_axis_name="s")
    return pl.kernel(
        _dispatch_body,
        out_type=[
            jax.ShapeDtypeStruct((BUF,), jnp.float32),
            jax.ShapeDtypeStruct((BUF, D), jnp.float32),
        ],
        mesh=mesh,
        compiler_params=_SC_PARAMS,
        scratch_types=[
            pltpu.VMEM((T,), jnp.int32),
            pltpu.VMEM((T,), jnp.float32),
            pltpu.VMEM((ROWS_W,), jnp.int32),
            pltpu.VMEM((ROWS_W,), jnp.float32),
            pltpu.VMEM((GCHUNK, D), jnp.float32),
            pltpu.VMEM((GCHUNK, D), jnp.float32),
            pltpu.SemaphoreType.DMA,
            pltpu.SemaphoreType.DMA,
            pltpu.SemaphoreType.DMA,
            pltpu.SemaphoreType.DMA,
            pltpu.SemaphoreType.DMA,
            pltpu.VMEM((16,), jnp.int32),
        ],
    )(d0, d1, s0, s1, xt, nact)


def _mlp_body(te_ref, na_ref, xs_ref, sco_ref, wg_ref, wu_ref, wd_ref,
              out_ref):
    i = pl.program_id(0)

    @pl.when(i < na_ref[0])
    def _():
        xv = xs_ref[...]
        g = jnp.dot(xv, wg_ref[0], preferred_element_type=jnp.float32)
        u = jnp.dot(xv, wu_ref[0], preferred_element_type=jnp.float32)
        h = g * jax.nn.sigmoid(g) * u
        out = jnp.dot(h, wd_ref[0], preferred_element_type=jnp.float32)
        out_ref[...] = out * sco_ref[...]


def _run_mlp(te, nact, xs, sco, W_gate, W_up, W_down):
    clamp = lambda i, na: jnp.minimum(i, na[0] - 1)
    grid_spec = pltpu.PrefetchScalarGridSpec(
        num_scalar_prefetch=2,
        grid=(NT,),
        in_specs=[
            pl.BlockSpec((TILE, D), lambda i, te, na: (clamp(i, na), 0)),
            pl.BlockSpec((TILE, 1), lambda i, te, na: (clamp(i, na), 0)),
            pl.BlockSpec((1, D, H), lambda i, te, na: (te[i], 0, 0)),
            pl.BlockSpec((1, D, H), lambda i, te, na: (te[i], 0, 0)),
            pl.BlockSpec((1, H, D), lambda i, te, na: (te[i], 0, 0)),
        ],
        out_specs=pl.BlockSpec((TILE, D),
                               lambda i, te, na: (clamp(i, na), 0)),
    )
    return pl.pallas_call(
        _mlp_body,
        grid_spec=grid_spec,
        out_shape=jax.ShapeDtypeStruct((BUF, D), jnp.float32),
    )(te, nact, xs, sco, W_gate, W_up, W_down)


def _combine_body(ys_hbm, d0_hbm, d1_hbm, y_hbm, i0_v, i1_v, a_v, b_v,
                  sem0, sem1):
    wid = lax.axis_index("s") * NC + lax.axis_index("c")
    base = wid * TOK_W
    pltpu.sync_copy(d0_hbm.at[pl.ds(base, TOK_W)], i0_v)
    pltpu.sync_copy(d1_hbm.at[pl.ds(base, TOK_W)], i1_v)
    g0 = pltpu.async_copy(ys_hbm.at[i0_v], a_v, sem0)
    g1 = pltpu.async_copy(ys_hbm.at[i1_v], b_v, sem1)
    g0.wait()
    g1.wait()

    def row_body(r, _):
        for c in range(D // 16):
            sl = pl.ds(c * 16, 16)
            a_v[r, sl] = a_v[r, sl] + b_v[r, sl]
        return 0

    lax.fori_loop(0, TOK_W, row_body, 0)
    pltpu.sync_copy(a_v, y_hbm.at[pl.ds(base, TOK_W)])


def _run_combine(ys, d0, d1):
    mesh = plsc.VectorSubcoreMesh(core_axis_name="c", subcore_axis_name="s")
    return pl.kernel(
        _combine_body,
        out_type=jax.ShapeDtypeStruct((T, D), jnp.float32),
        mesh=mesh,
        compiler_params=_SC_PARAMS,
        scratch_types=[
            pltpu.VMEM((TOK_W,), jnp.int32),
            pltpu.VMEM((TOK_W,), jnp.int32),
            pltpu.VMEM((TOK_W, D), jnp.float32),
            pltpu.VMEM((TOK_W, D), jnp.float32),
            pltpu.SemaphoreType.DMA,
            pltpu.SemaphoreType.DMA,
        ],
    )(ys, d0, d1)


@functools.partial(jax.jit, static_argnames=())
def kernel(x, Wg1, Wg2, W_gate, W_up, W_down):
    orig_shape = x.shape[:-1]
    xt = x.reshape(T, D)

    tril = (lax.broadcasted_iota(jnp.int32, (CH, CH), 1)
            < lax.broadcasted_iota(jnp.int32, (CH, CH), 0)
            ).astype(jnp.bfloat16)

    (d0c, d1c, s0c, s1c, load, imp, loss, te, nact) = _run_gate(
        xt, Wg1, Wg2, tril)
    d0 = d0c.reshape(T)
    d1 = d1c.reshape(T)

    sco, xs = _run_dispatch(xt, d0, d1, s0c.reshape(T), s1c.reshape(T),
                            jnp.broadcast_to(nact.reshape(1), (16,)))
    ys = _run_mlp(te.reshape(NT), nact.reshape(1), xs, sco.reshape(BUF, 1),
                  W_gate, W_up, W_down)
    y2d = _run_combine(ys, d0, d1)

    y = y2d.reshape(orig_shape + (D,))
    return (y, loss.reshape(()), load.reshape(E), imp.reshape(E))


# speedup vs baseline: 1.0068x; 1.0068x over previous
import functools

import jax
import jax.numpy as jnp
from jax import lax
from jax.experimental import pallas as pl
from jax.experimental.pallas import tpu as pltpu
from jax.experimental.pallas import tpu_sc as plsc

_SC_PARAMS = pltpu.CompilerParams(needs_layout_passes=False)

E = 64
K = 2
D = 768
H = 768
T = 2048
TILE = 128
BUF = T * K + E * TILE
NT = BUF // TILE
CH = 512
BALANCE_W = 0.01

NC = 2
NS = 16
NW = NC * NS
ROWS_W = BUF // NW
GCHUNK = 64
TOK_W = T // NW


def _shift_lanes(v, sh):
    return jnp.concatenate([jnp.zeros((1, sh), v.dtype), v[:, : E - sh]], axis=1)


def _cv_sq(v):
    n = v.shape[1]
    mean = jnp.sum(v, axis=1, keepdims=True) / n
    var = jnp.sum((v - mean) ** 2, axis=1, keepdims=True) / (n - 1)
    return var / (mean * mean + 1e-10)


def _gate_body(x_ref, wg1_ref, wg2_ref, tril_ref,
               d0_ref, d1_ref, s0_ref, s1_ref,
               load_ref, imp_ref, loss_ref, te_ref, na_ref):
    xv = x_ref[...]
    a = jnp.tanh(jnp.dot(xv, wg1_ref[...],
                         preferred_element_type=jnp.float32))
    logits = jnp.dot(a, wg2_ref[...], preferred_element_type=jnp.float32)

    iota_f = lax.broadcasted_iota(jnp.int32, (T, E), 1).astype(jnp.float32)
    m1 = jnp.max(logits, axis=1, keepdims=True)
    i1 = jnp.min(jnp.where(logits == m1, iota_f, float(E)), axis=1,
                 keepdims=True)
    sel1 = iota_f == i1
    masked = jnp.where(sel1, -jnp.inf, logits)
    m2 = jnp.max(masked, axis=1, keepdims=True)
    i2 = jnp.min(jnp.where(masked == m2, iota_f, float(E)), axis=1,
                 keepdims=True)
    sel2 = iota_f == i2

    z = jnp.exp(m2 - m1)
    s0 = 1.0 / (1.0 + z)
    s1 = z / (1.0 + z)

    sel1f = sel1.astype(jnp.float32)
    sel2f = sel2.astype(jnp.float32)
    pair_cnt = sel1f + sel2f
    counts = jnp.sum(pair_cnt, axis=0, keepdims=True)
    imp = jnp.sum(sel1f * s0 + sel2f * s1, axis=0, keepdims=True)

    trilv = tril_ref[...]
    carry = jnp.zeros((1, E), jnp.float32)
    chunks = []
    for k in range(T // CH):
        ak = pair_cnt[k * CH:(k + 1) * CH, :]
        ck = jax.lax.dot_general(trilv, ak.astype(jnp.bfloat16),
                                 (((1,), (0,)), ((), ())),
                                 preferred_element_type=jnp.float32) + carry
        chunks.append(ck)
        carry = carry + jnp.sum(ak, axis=0, keepdims=True)
    csum = jnp.concatenate(chunks, axis=0)

    rank0 = jnp.sum(jnp.where(sel1, csum, 0.0), axis=1, keepdims=True)
    rank1 = jnp.sum(jnp.where(sel2, csum, 0.0), axis=1, keepdims=True)

    padded = jnp.ceil(counts / TILE) * TILE
    inc = padded
    for sh in (1, 2, 4, 8, 16, 32):
        inc = inc + _shift_lanes(inc, sh)
    pstart = inc - padded

    pstart_b = jnp.broadcast_to(pstart, (T, E))
    p0 = jnp.sum(jnp.where(sel1, pstart_b, 0.0), axis=1, keepdims=True)
    p1 = jnp.sum(jnp.where(sel2, pstart_b, 0.0), axis=1, keepdims=True)
    d0_ref[...] = (p0 + rank0).astype(jnp.int32)
    d1_ref[...] = (p1 + rank1).astype(jnp.int32)
    s0_ref[...] = s0
    s1_ref[...] = s1
    load_ref[...] = counts
    imp_ref[...] = imp

    total = jnp.sum(padded, axis=1, keepdims=True)
    na_ref[...] = (total / TILE).astype(jnp.int32)

    starts = (lax.broadcasted_iota(jnp.int32, (NT, E), 0)
              .astype(jnp.float32) * TILE)
    ps_nt = jnp.broadcast_to(pstart, (NT, E))
    te = jnp.sum((ps_nt <= starts).astype(jnp.float32), axis=1,
                 keepdims=True) - 1.0
    te_ref[...] = te.astype(jnp.int32)

    loss_ref[...] = BALANCE_W * (_cv_sq(imp) + _cv_sq(counts))


def _run_gate(xt, Wg1, Wg2, tril):
    f32 = jnp.float32
    outs = pl.pallas_call(
        _gate_body,
        out_shape=[
            jax.ShapeDtypeStruct((T, 1), jnp.int32),
            jax.ShapeDtypeStruct((T, 1), jnp.int32),
            jax.ShapeDtypeStruct((T, 1), f32),
            jax.ShapeDtypeStruct((T, 1), f32),
            jax.ShapeDtypeStruct((1, E), f32),
            jax.ShapeDtypeStruct((1, E), f32),
            jax.ShapeDtypeStruct((1, 1), f32),
            jax.ShapeDtypeStruct((NT, 1), jnp.int32),
            jax.ShapeDtypeStruct((1, 1), jnp.int32),
        ],
    )(xt, Wg1, Wg2, tril)
    return outs


def _dispatch_body(d0_hbm, d1_hbm, s0_hbm, s1_hbm, xt_hbm,
                   sco_hbm, xs_hbm,
                   d_v, s_v, src_v, sco_v, rows0_v, rows1_v,
                   sg0, sg1, sw0, sw1, ssco):
    wid = lax.axis_index("s") * NC + lax.axis_index("c")
    base = wid * ROWS_W
    iota16 = lax.broadcasted_iota(jnp.int32, (16,), 0)
    z16f = jnp.zeros((16,), jnp.float32)

    def init_body(j, _):
        src_v[pl.ds(j * 16, 16)] = (base + iota16 + j * 16) & (T - 1)
        sco_v[pl.ds(j * 16, 16)] = z16f
        return 0

    lax.fori_loop(0, ROWS_W // 16, init_body, 0)

    def scatter_pass(dref, sref):
        pltpu.sync_copy(dref, d_v)
        pltpu.sync_copy(sref, s_v)

        def body(i, _):
            idx = d_v[pl.ds(i * 16, 16)]
            loc = idx - base
            mask = (idx >= base) & (idx < base + ROWS_W)
            plsc.store_scatter(src_v, [loc], iota16 + i * 16, mask=mask)
            plsc.store_scatter(sco_v, [loc], s_v[pl.ds(i * 16, 16)],
                               mask=mask)
            return 0

        lax.fori_loop(0, T // 16, body, 0)

    scatter_pass(d0_hbm, s0_hbm)
    scatter_pass(d1_hbm, s1_hbm)
    wsco = pltpu.async_copy(sco_v, sco_hbm.at[pl.ds(base, ROWS_W)], ssco)

    nch = ROWS_W // GCHUNK
    bufs = (rows0_v, rows1_v)
    gsems = (sg0, sg1)
    wsems = (sw0, sw1)
    gathers = [None] * nch
    writes = [None] * nch
    for c in range(nch):
        b = c % 2
        if c >= 2:
            writes[c - 2].wait()
        gathers[c] = pltpu.async_copy(
            xt_hbm.at[src_v.at[pl.ds(c * GCHUNK, GCHUNK)]], bufs[b], gsems[b])
        if c >= 1:
            gathers[c - 1].wait()
            writes[c - 1] = pltpu.async_copy(
                bufs[(c - 1) % 2],
                xs_hbm.at[pl.ds(base + (c - 1) * GCHUNK, GCHUNK)],
                wsems[(c - 1) % 2])
    gathers[nch - 1].wait()
    writes[nch - 1] = pltpu.async_copy(
        bufs[(nch - 1) % 2],
        xs_hbm.at[pl.ds(base + (nch - 1) * GCHUNK, GCHUNK)],
        wsems[(nch - 1) % 2])
    for c in range(max(0, nch - 2), nch):
        writes[c].wait()
    wsco.wait()


def _run_dispatch(xt, d0, d1, s0, s1):
    mesh = plsc.VectorSubcoreMesh(core_axis_name="c", subcore_axis_name="s")
    return pl.kernel(
        _dispatch_body,
        out_type=[
            jax.ShapeDtypeStruct((BUF,), jnp.float32),
            jax.ShapeDtypeStruct((BUF, D), jnp.float32),
        ],
        mesh=mesh,
        compiler_params=_SC_PARAMS,
        scratch_types=[
            pltpu.VMEM((T,), jnp.int32),
            pltpu.VMEM((T,), jnp.float32),
            pltpu.VMEM((ROWS_W,), jnp.int32),
            pltpu.VMEM((ROWS_W,), jnp.float32),
            pltpu.VMEM((GCHUNK, D), jnp.float32),
            pltpu.VMEM((GCHUNK, D), jnp.float32),
            pltpu.SemaphoreType.DMA,
            pltpu.SemaphoreType.DMA,
            pltpu.SemaphoreType.DMA,
            pltpu.SemaphoreType.DMA,
            pltpu.SemaphoreType.DMA,
        ],
    )(d0, d1, s0, s1, xt)


def _mlp_body(te_ref, na_ref, xs_ref, sco_ref, wg_ref, wu_ref, wd_ref,
              out_ref):
    i = pl.program_id(0)

    @pl.when(i < na_ref[0])
    def _():
        xv = xs_ref[...]
        g = jnp.dot(xv, wg_ref[0], preferred_element_type=jnp.float32)
        u = jnp.dot(xv, wu_ref[0], preferred_element_type=jnp.float32)
        h = g * jax.nn.sigmoid(g) * u
        out = jnp.dot(h, wd_ref[0], preferred_element_type=jnp.float32)
        out_ref[...] = out * sco_ref[...]


def _run_mlp(te, nact, xs, sco, W_gate, W_up, W_down):
    clamp = lambda i, na: jnp.minimum(i, na[0] - 1)
    grid_spec = pltpu.PrefetchScalarGridSpec(
        num_scalar_prefetch=2,
        grid=(NT,),
        in_specs=[
            pl.BlockSpec((TILE, D), lambda i, te, na: (clamp(i, na), 0)),
            pl.BlockSpec((TILE, 1), lambda i, te, na: (clamp(i, na), 0)),
            pl.BlockSpec((1, D, H), lambda i, te, na: (te[i], 0, 0)),
            pl.BlockSpec((1, D, H), lambda i, te, na: (te[i], 0, 0)),
            pl.BlockSpec((1, H, D), lambda i, te, na: (te[i], 0, 0)),
        ],
        out_specs=pl.BlockSpec((TILE, D),
                               lambda i, te, na: (clamp(i, na), 0)),
    )
    return pl.pallas_call(
        _mlp_body,
        grid_spec=grid_spec,
        out_shape=jax.ShapeDtypeStruct((BUF, D), jnp.float32),
    )(te, nact, xs, sco, W_gate, W_up, W_down)


def _combine_body(ys_hbm, d0_hbm, d1_hbm, y_hbm, i0_v, i1_v, a_v, b_v,
                  sem0, sem1):
    wid = lax.axis_index("s") * NC + lax.axis_index("c")
    base = wid * TOK_W
    pltpu.sync_copy(d0_hbm.at[pl.ds(base, TOK_W)], i0_v)
    pltpu.sync_copy(d1_hbm.at[pl.ds(base, TOK_W)], i1_v)
    g0 = pltpu.async_copy(ys_hbm.at[i0_v], a_v, sem0)
    g1 = pltpu.async_copy(ys_hbm.at[i1_v], b_v, sem1)
    g0.wait()
    g1.wait()

    def row_body(r, _):
        for c in range(D // 16):
            sl = pl.ds(c * 16, 16)
            a_v[r, sl] = a_v[r, sl] + b_v[r, sl]
        return 0

    lax.fori_loop(0, TOK_W, row_body, 0)
    pltpu.sync_copy(a_v, y_hbm.at[pl.ds(base, TOK_W)])


def _run_combine(ys, d0, d1):
    mesh = plsc.VectorSubcoreMesh(core_axis_name="c", subcore_axis_name="s")
    return pl.kernel(
        _combine_body,
        out_type=jax.ShapeDtypeStruct((T, D), jnp.float32),
        mesh=mesh,
        compiler_params=_SC_PARAMS,
        scratch_types=[
            pltpu.VMEM((TOK_W,), jnp.int32),
            pltpu.VMEM((TOK_W,), jnp.int32),
            pltpu.VMEM((TOK_W, D), jnp.float32),
            pltpu.VMEM((TOK_W, D), jnp.float32),
            pltpu.SemaphoreType.DMA,
            pltpu.SemaphoreType.DMA,
        ],
    )(ys, d0, d1)


@functools.partial(jax.jit, static_argnames=())
def kernel(x, Wg1, Wg2, W_gate, W_up, W_down):
    orig_shape = x.shape[:-1]
    xt = x.reshape(T, D)

    tril = (lax.broadcasted_iota(jnp.int32, (CH, CH), 1)
            < lax.broadcasted_iota(jnp.int32, (CH, CH), 0)
            ).astype(jnp.bfloat16)

    (d0c, d1c, s0c, s1c, load, imp, loss, te, nact) = _run_gate(
        xt, Wg1, Wg2, tril)
    d0 = d0c.reshape(T)
    d1 = d1c.reshape(T)

    sco, xs = _run_dispatch(xt, d0, d1, s0c.reshape(T), s1c.reshape(T))
    ys = _run_mlp(te.reshape(NT), nact.reshape(1), xs, sco.reshape(BUF, 1),
                  W_gate, W_up, W_down)
    y2d = _run_combine(ys, d0, d1)

    y = y2d.reshape(orig_shape + (D,))
    return (y, loss.reshape(()), load.reshape(E), imp.reshape(E))
